# baseline (speedup 1.0000x reference)
import functools
import math

import numpy as np

import jax
import jax.numpy as jnp
from jax import lax
from jax.experimental import pallas as pl
from jax.experimental.pallas import tpu as pltpu
from jax.experimental.pallas import tpu_sc as plsc

B = 1024
V = 100000
KS = (1, 5, 10, 20)

NC = 2
NS = 16
NW = NC * NS
L = 16

RPW = B // NW
CH = 20000
NCH = V // CH
TOTAL = RPW * NCH
NBUF = 4
CVECS = CH // L

NMET = 9
MSTRIDE = NMET * L

_mesh = plsc.VectorSubcoreMesh(core_axis_name="c", subcore_axis_name="s")


def _rank_body(scores_hbm, labels_hbm, table_hbm, out_hbm,
               lab_ref, idx_ref, pred_ref, rank_ref, tab_ref, met_ref,
               b0, b1, b2, b3, s0, s1, s2, s3, gsem):
    bufs = (b0, b1, b2, b3)
    sems = (s0, s1, s2, s3)
    cid = lax.axis_index("c")
    sid = lax.axis_index("s")
    wid = sid * NC + cid
    base_row = wid * RPW
    flat_base = base_row * V
    lane = lax.iota(jnp.int32, L)

    pltpu.sync_copy(labels_hbm.at[pl.ds(base_row, RPW)], lab_ref)
    pltpu.sync_copy(table_hbm, tab_ref)

    for g in range(RPW // L):
        lab_v = lab_ref[pl.ds(g * L, L)]
        row_v = lane + (base_row + g * L)
        idx_ref[pl.ds(g * L, L)] = row_v * V + lab_v
    pltpu.async_copy(scores_hbm.at[idx_ref], pred_ref, gsem).wait()

    for b in range(NBUF):
        pltpu.async_copy(
            scores_hbm.at[pl.ds(flat_base + b * CH, CH)], bufs[b], sems[b])

    zf = jnp.zeros((L,), jnp.float32)
    zi = jnp.zeros((L,), jnp.int32)

    def chunk_step(t_base, carry):
        count_v, ranks_v = carry
        for b in range(NBUF):
            t = t_base + b
            buf, sem = bufs[b], sems[b]
            pltpu.make_async_copy(scores_hbm.at[pl.ds(0, CH)], buf, sem).wait()

            r_local = t // NCH
            pred_v = plsc.load_gather(
                pred_ref, [jnp.full((L,), r_local, jnp.int32)])

            def inner(j, cv):
                x = buf[pl.ds(j * L, L)]
                return cv + plsc.all_reduce_population_count(x > pred_v)

            count_v = lax.fori_loop(0, CVECS, inner, count_v, unroll=8)

            @pl.when(t + NBUF < TOTAL)
            def _():
                pltpu.async_copy(
                    scores_hbm.at[pl.ds(flat_base + (t + NBUF) * CH, CH)],
                    buf, sem)

            is_end = (t % NCH) == (NCH - 1)
            rank_s = jnp.sum(count_v).astype(jnp.float32) * (1.0 / L)
            ranks_v = jnp.where(
                jnp.logical_and(is_end, lane == (r_local % L)),
                ranks_v + rank_s, ranks_v)
            count_v = jnp.where(is_end, zi, count_v)

            g_end = jnp.logical_and(is_end, (r_local % L) == (L - 1))

            @pl.when(g_end)
            def _():
                rank_ref[pl.ds((r_local // L) * L, L)] = ranks_v

            ranks_v = jnp.where(g_end, zf, ranks_v)
        return count_v, ranks_v

    pl.loop(0, TOTAL, step=NBUF, init_carry=(zi, zf))(chunk_step)

    acc = [zf] * NMET
    for g in range(RPW // L):
        r_v = rank_ref[pl.ds(g * L, L)]
        t_idx = jnp.minimum(r_v.astype(jnp.int32), 31)
        dcg_v = plsc.load_gather(tab_ref, [t_idx])
        mi = 0
        for k in KS:
            ind = (r_v < float(k)).astype(jnp.float32)
            acc[mi] = acc[mi] + dcg_v * ind
            acc[mi + 1] = acc[mi + 1] + ind
            mi += 2
        acc[mi] = acc[mi] + 1.0 / (r_v + 1.0)
    for i in range(NMET):
        met_ref[pl.ds(i * L, L)] = acc[i]
    pltpu.sync_copy(met_ref, out_hbm.at[pl.ds(wid * MSTRIDE, MSTRIDE)])


_sc_params = pltpu.CompilerParams(needs_layout_passes=False)

_rank_call = pl.kernel(
    _rank_body,
    out_type=jax.ShapeDtypeStruct((NW * MSTRIDE,), jnp.float32),
    mesh=_mesh,
    compiler_params=_sc_params,
    scratch_types=[
        pltpu.VMEM((RPW,), jnp.int32),
        pltpu.VMEM((RPW,), jnp.int32),
        pltpu.VMEM((RPW,), jnp.float32),
        pltpu.VMEM((RPW,), jnp.float32),
        pltpu.VMEM((32,), jnp.float32),
        pltpu.VMEM((MSTRIDE,), jnp.float32),
        pltpu.VMEM((CH,), jnp.float32),
        pltpu.VMEM((CH,), jnp.float32),
        pltpu.VMEM((CH,), jnp.float32),
        pltpu.VMEM((CH,), jnp.float32),
        pltpu.SemaphoreType.DMA,
        pltpu.SemaphoreType.DMA,
        pltpu.SemaphoreType.DMA,
        pltpu.SemaphoreType.DMA,
        pltpu.SemaphoreType.DMA,
    ],
)


def _combine_body(parts_hbm, out_hbm, pbuf, obuf, csem):
    cid = lax.axis_index("c")
    sid = lax.axis_index("s")
    wid = sid * NC + cid

    @pl.when(wid == 0)
    def _():
        pltpu.sync_copy(parts_hbm, pbuf)
        lane = lax.iota(jnp.int32, L)
        out_v = jnp.zeros((L,), jnp.float32)
        for i in range(NMET):
            def body(w, a, i=i):
                return a + pbuf[pl.ds(w * MSTRIDE + i * L, L)]
            acc = lax.fori_loop(0, NW, body, jnp.zeros((L,), jnp.float32))
            s = jnp.sum(acc) * (1.0 / B)
            out_v = jnp.where(lane == i, s, out_v)
        obuf[...] = out_v
        pltpu.sync_copy(obuf, out_hbm)


_combine_call = pl.kernel(
    _combine_body,
    out_type=jax.ShapeDtypeStruct((L,), jnp.float32),
    mesh=_mesh,
    compiler_params=_sc_params,
    scratch_types=[
        pltpu.VMEM((NW * MSTRIDE,), jnp.float32),
        pltpu.VMEM((L,), jnp.float32),
        pltpu.SemaphoreType.DMA,
    ],
)

_TABLE = np.array([1.0 / math.log2(i + 2.0) for i in range(32)],
                  dtype=np.float32)


def kernel(scores, labels):
    flat = scores.reshape(-1)
    parts = _rank_call(flat, labels, _TABLE)
    out16 = _combine_call(parts)
    return out16[:NMET]
